# Initial kernel scaffold; baseline (speedup 1.0000x reference)
import jax
import jax.numpy as jnp
from jax.experimental import pallas as pl


def kernel(fixed_input_features, gemm_edges, We0, be0, We1, be1, We2, be2, Wd0, bd0, Wd1, bd1, Wd2, bd2, Wf, bf):
    raise NotImplementedError("write your pallas kernel here")


# speedup vs baseline: 19.2872x; 19.2872x over previous
import functools

import jax
import jax.numpy as jnp
from jax import lax
from jax.experimental import pallas as pl
from jax.experimental.pallas import tpu as pltpu
from jax.experimental.pallas import tpu_sc as plsc

_NC = 2
_NS = 16
_NW = _NC * _NS

_LANES = 128
_IDX_CHUNK = 128
_HALF = 512
_SUPER = 1024


def _sc_gather(table, idx2d):
    M = idx2d.shape[0] * idx2d.shape[1]
    per_w = M // _NW
    supers = per_w // _SUPER
    mesh = plsc.VectorSubcoreMesh(
        core_axis_name="c", subcore_axis_name="s",
        num_cores=_NC, num_subcores=_NS)

    @functools.partial(
        pl.kernel,
        out_type=jax.ShapeDtypeStruct((M, _LANES), jnp.float32),
        mesh=mesh,
        scratch_types=[
            pltpu.VMEM((_SUPER // _IDX_CHUNK, _IDX_CHUNK), jnp.int32),
            pltpu.VMEM((_HALF, _LANES), jnp.float32),
            pltpu.SemaphoreType.DMA,
        ],
    )
    def gather_kernel(table_hbm, idx_hbm, out_hbm, idx_v, rows_v, sem):
        wid = lax.axis_index("s") * _NC + lax.axis_index("c")
        base = wid * per_w

        def body(i, carry):
            irow0 = pl.multiple_of((base + i * _SUPER) // _IDX_CHUNK, 8)
            pltpu.sync_copy(idx_hbm.at[pl.ds(irow0, _SUPER // _IDX_CHUNK)], idx_v)
            for half in range(_SUPER // _HALF):
                row0 = pl.multiple_of(base + i * _SUPER + half * _HALF, _HALF)
                copies = []
                for j in range(_HALF // _IDX_CHUNK):
                    copies.append(pltpu.async_copy(
                        table_hbm.at[idx_v.at[half * (_HALF // _IDX_CHUNK) + j]],
                        rows_v.at[pl.ds(j * _IDX_CHUNK, _IDX_CHUNK)],
                        sem))
                for cp in copies:
                    cp.wait()
                pltpu.sync_copy(rows_v, out_hbm.at[pl.ds(row0, _HALF)])
            return carry

        lax.fori_loop(0, supers, body, 0)

    return gather_kernel(table, idx2d)


def _tc_conv(xp, g, Ws, bias, skip, act, narrow_out):
    E = xp.shape[0]
    C, F = Ws[0].shape
    R = 2048
    grid = (E // R,)
    nb = E // R
    row_spec = pl.BlockSpec((R, _LANES), lambda i: (i, 0))
    g_specs = [pl.BlockSpec((R, _LANES), lambda i, k=k: (i + k * nb, 0))
               for k in range(4)]
    w_specs = [pl.BlockSpec((C, F), lambda i: (0, 0)) for _ in range(5)]
    b_spec = pl.BlockSpec((1, F), lambda i: (0, 0))
    out_w = F if narrow_out else _LANES
    out_spec = pl.BlockSpec((R, out_w), lambda i: (i, 0))

    def body(x_ref, a_ref, b_ref, c_ref, d_ref, w0, w1, w2, w3, w4,
             bias_ref, *rest):
        out_ref = rest[-1]
        xx = x_ref[...][:, :C]
        a = a_ref[...][:, :C]
        b = b_ref[...][:, :C]
        c = c_ref[...][:, :C]
        d = d_ref[...][:, :C]
        dot = functools.partial(
            jnp.dot, precision=lax.Precision.HIGHEST,
            preferred_element_type=jnp.float32)
        h = dot(xx, w0[...])
        h += dot(jnp.abs(a - c), w1[...])
        h += dot(a + c, w2[...])
        h += dot(jnp.abs(b - d), w3[...])
        h += dot(b + d, w4[...])
        h += bias_ref[...]
        if act:
            h = jnp.where(h >= 0, h, 0.1 * h)
        if len(rest) == 2:
            h += rest[0][...][:, :F]
        if not narrow_out:
            h = jnp.pad(h, ((0, 0), (0, _LANES - F)))
        out_ref[...] = h

    in_specs = [row_spec] + g_specs + w_specs + [b_spec]
    args = [xp, g, g, g, g] + list(Ws) + [bias.reshape(1, F)]
    if skip is not None:
        in_specs.append(row_spec)
        args.append(skip)
    return pl.pallas_call(
        body,
        grid=grid,
        in_specs=in_specs,
        out_specs=out_spec,
        out_shape=jax.ShapeDtypeStruct((E, out_w), jnp.float32),
    )(*args)


def _layer(hp, idx2d, W, bias, skip, act, narrow_out=False):
    C = W.shape[0] // 5
    Ws = [W[k * C:(k + 1) * C] for k in range(5)]
    g = _sc_gather(hp, idx2d)
    return _tc_conv(hp, g, Ws, bias, skip, act, narrow_out)


def kernel(fixed_input_features, gemm_edges, We0, be0, We1, be1, We2, be2,
           Wd0, bd0, Wd1, bd1, Wd2, bd2, Wf, bf):
    E = fixed_input_features.shape[0]
    idx2d = gemm_edges.T.reshape(4 * E // _IDX_CHUNK, _IDX_CHUNK)

    x0p = jnp.pad(fixed_input_features, ((0, 0), (0, _LANES - 6)))
    s1 = _layer(x0p, idx2d, We0, be0, None, True)
    s2 = _layer(s1, idx2d, We1, be1, None, True)
    h = _layer(s2, idx2d, We2, be2, None, True)
    h = _layer(h, idx2d, Wd0, bd0, s2, True)
    h = _layer(h, idx2d, Wd1, bd1, s1, True)
    h = _layer(h, idx2d, Wd2, bd2, x0p, True)
    return _layer(h, idx2d, Wf, bf, None, False, narrow_out=True)


# speedup vs baseline: 27.4024x; 1.4208x over previous
import functools

import jax
import jax.numpy as jnp
from jax import lax
from jax.experimental import pallas as pl
from jax.experimental.pallas import tpu as pltpu
from jax.experimental.pallas import tpu_sc as plsc

_NC = 2
_NS = 16
_NW = _NC * _NS

_LANES = 128
_IDX_CHUNK = 128
_HALF = 512
_SUPER = 1024


def _sc_gather(table, idx2d, width):
    M = idx2d.shape[0] * idx2d.shape[1]
    per_w = M // _NW
    supers = per_w // _SUPER
    mesh = plsc.VectorSubcoreMesh(
        core_axis_name="c", subcore_axis_name="s",
        num_cores=_NC, num_subcores=_NS)

    @functools.partial(
        pl.kernel,
        out_type=jax.ShapeDtypeStruct((M, width), jnp.float32),
        mesh=mesh,
        scratch_types=[
            pltpu.VMEM((_SUPER // _IDX_CHUNK, _IDX_CHUNK), jnp.int32),
            pltpu.VMEM((_HALF, _LANES), jnp.float32),
            pltpu.SemaphoreType.DMA,
        ],
    )
    def gather_kernel(table_hbm, idx_hbm, out_hbm, idx_v, rows_v, sem):
        wid = lax.axis_index("s") * _NC + lax.axis_index("c")
        base = wid * per_w

        def body(i, carry):
            irow0 = pl.multiple_of((base + i * _SUPER) // _IDX_CHUNK, 8)
            pltpu.sync_copy(idx_hbm.at[pl.ds(irow0, _SUPER // _IDX_CHUNK)], idx_v)
            for half in range(_SUPER // _HALF):
                row0 = pl.multiple_of(base + i * _SUPER + half * _HALF, _HALF)
                copies = []
                for j in range(_HALF // _IDX_CHUNK):
                    copies.append(pltpu.async_copy(
                        table_hbm.at[idx_v.at[half * (_HALF // _IDX_CHUNK) + j]],
                        rows_v.at[pl.ds(j * _IDX_CHUNK, _IDX_CHUNK)],
                        sem))
                for cp in copies:
                    cp.wait()
                if width == _LANES:
                    pltpu.sync_copy(rows_v, out_hbm.at[pl.ds(row0, _HALF)])
                else:
                    pltpu.sync_copy(rows_v.at[:, pl.ds(0, width)],
                                    out_hbm.at[pl.ds(row0, _HALF)])
            return carry

        lax.fori_loop(0, supers, body, 0)

    return gather_kernel(table, idx2d)


def _tc_conv(xp, g, Ws, bias, skip, act, narrow_out):
    E = xp.shape[0]
    C, F = Ws[0].shape
    gw = g.shape[1]
    R = 2048
    grid = (E // R,)
    nb = E // R
    row_spec = pl.BlockSpec((R, _LANES), lambda i: (i, 0))
    g_specs = [pl.BlockSpec((R, gw), lambda i, k=k: (i + k * nb, 0))
               for k in range(4)]
    w_specs = [pl.BlockSpec((C, F), lambda i: (0, 0)) for _ in range(5)]
    b_spec = pl.BlockSpec((1, F), lambda i: (0, 0))
    out_w = F if narrow_out else _LANES
    out_spec = pl.BlockSpec((R, out_w), lambda i: (i, 0))

    def body(x_ref, a_ref, b_ref, c_ref, d_ref, w0, w1, w2, w3, w4,
             bias_ref, *rest):
        out_ref = rest[-1]
        xx = x_ref[...][:, :C]
        a = a_ref[...][:, :C]
        b = b_ref[...][:, :C]
        c = c_ref[...][:, :C]
        d = d_ref[...][:, :C]
        dot = functools.partial(
            jnp.dot, preferred_element_type=jnp.float32)
        h = dot(xx, w0[...])
        h += dot(jnp.abs(a - c), w1[...])
        h += dot(a + c, w2[...])
        h += dot(jnp.abs(b - d), w3[...])
        h += dot(b + d, w4[...])
        h += bias_ref[...]
        if act:
            h = jnp.where(h >= 0, h, 0.1 * h)
        if len(rest) == 2:
            h += rest[0][...][:, :F]
        if not narrow_out:
            h = jnp.pad(h, ((0, 0), (0, _LANES - F)))
        out_ref[...] = h

    in_specs = [row_spec] + g_specs + w_specs + [b_spec]
    args = [xp, g, g, g, g] + list(Ws) + [bias.reshape(1, F)]
    if skip is not None:
        in_specs.append(row_spec)
        args.append(skip)
    return pl.pallas_call(
        body,
        grid=grid,
        in_specs=in_specs,
        out_specs=out_spec,
        out_shape=jax.ShapeDtypeStruct((E, out_w), jnp.float32),
    )(*args)


def _layer(hp, idx2d, W, bias, skip, act, narrow_out=False):
    C = W.shape[0] // 5
    Ws = [W[k * C:(k + 1) * C] for k in range(5)]
    g = _sc_gather(hp, idx2d, _LANES)
    return _tc_conv(hp, g, Ws, bias, skip, act, narrow_out)


def kernel(fixed_input_features, gemm_edges, We0, be0, We1, be1, We2, be2,
           Wd0, bd0, Wd1, bd1, Wd2, bd2, Wf, bf):
    E = fixed_input_features.shape[0]
    idx2d = gemm_edges.T.reshape(4 * E // _IDX_CHUNK, _IDX_CHUNK)

    x0p = jnp.pad(fixed_input_features, ((0, 0), (0, _LANES - 6)))
    s1 = _layer(x0p, idx2d, We0, be0, None, True)
    s2 = _layer(s1, idx2d, We1, be1, None, True)
    h = _layer(s2, idx2d, We2, be2, None, True)
    h = _layer(h, idx2d, Wd0, bd0, s2, True)
    h = _layer(h, idx2d, Wd1, bd1, s1, True)
    h = _layer(h, idx2d, Wd2, bd2, x0p, True)
    return _layer(h, idx2d, Wf, bf, None, False, narrow_out=True)
